# baseline (speedup 1.0000x reference)
import functools

import jax
import jax.numpy as jnp
from jax import lax
from jax.experimental import pallas as pl
from jax.experimental.pallas import tpu as pltpu
from jax.experimental.pallas import tpu_sc as plsc

N = 10000
D = 128
NC = 2
NS = 16
NW = NC * NS
NPAD = 10112
ROWS_PER_TILE = NPAD // NS
CHUNK = 128
NBUF = 2
ROUNDS = 40
EPW = ROUNDS * NBUF * CHUNK
EPAD = NW * EPW

_MESH = plsc.VectorSubcoreMesh(core_axis_name="c", subcore_axis_name="s")


def _agg_body(with_cnt, *refs):
    if with_cnt:
        (h_hbm, eidx_hbm, zrows_hbm, ones_hbm, psum_hbm, cnt_hbm,
         idx0, idx1, *bufs) = refs
    else:
        (h_hbm, eidx_hbm, zrows_hbm, psum_hbm, idx0, idx1, *bufs) = refs
    rows = bufs[:NBUF]
    gsem = bufs[NBUF:2 * NBUF]
    ssem = bufs[2 * NBUF:3 * NBUF]
    acc_sh = bufs[3 * NBUF]
    c = lax.axis_index("c")
    s = lax.axis_index("s")
    wid = s * NC + c
    rs = s * ROWS_PER_TILE
    idxb = [idx0, idx1]
    ebase = (wid * ROUNDS) * (2 * NBUF)

    pltpu.sync_copy(zrows_hbm, acc_sh.at[pl.ds(rs, ROWS_PER_TILE)])
    plsc.subcore_barrier()

    pend_g = [None] * NBUF
    pend_s = [None] * NBUF
    for r in range(ROUNDS):
        iv = idxb[r & 1]
        pltpu.sync_copy(eidx_hbm.at[pl.ds(ebase + r * 2 * NBUF, 2 * NBUF)], iv)
        for b in range(NBUF):
            if pend_s[b] is not None:
                pend_s[b].wait()
            pend_g[b] = pltpu.async_copy(h_hbm.at[iv.at[b]], rows[b], gsem[b])
        for b in range(NBUF):
            pend_g[b].wait()
            pend_s[b] = pltpu.async_copy(rows[b], acc_sh.at[iv.at[NBUF + b]],
                                         ssem[b], add=True)
    for b in range(NBUF):
        pend_s[b].wait()
    plsc.subcore_barrier()

    pltpu.sync_copy(acc_sh.at[pl.ds(rs, ROWS_PER_TILE)],
                    psum_hbm.at[c, pl.ds(rs, ROWS_PER_TILE)])

    if with_cnt:
        plsc.subcore_barrier()
        pltpu.sync_copy(zrows_hbm, acc_sh.at[pl.ds(rs, ROWS_PER_TILE)])
        pltpu.sync_copy(ones_hbm, rows[0])
        plsc.subcore_barrier()

        pend = [None] * NBUF
        for r in range(ROUNDS):
            iv = idxb[r & 1]
            pltpu.sync_copy(
                eidx_hbm.at[pl.ds(ebase + r * 2 * NBUF, 2 * NBUF)], iv)
            for b in range(NBUF):
                if pend[b] is not None:
                    pend[b].wait()
                pend[b] = pltpu.async_copy(rows[0], acc_sh.at[iv.at[NBUF + b]],
                                           ssem[b], add=True)
        for b in range(NBUF):
            pend[b].wait()
        plsc.subcore_barrier()
        pltpu.sync_copy(acc_sh.at[pl.ds(rs, ROWS_PER_TILE)],
                        cnt_hbm.at[c, pl.ds(rs, ROWS_PER_TILE)])


def _make_agg(with_cnt):
    out_type = [jax.ShapeDtypeStruct((NC, NPAD, D), jnp.float32)]
    if with_cnt:
        out_type.append(jax.ShapeDtypeStruct((NC, NPAD, D), jnp.float32))
    scratch = [
        pltpu.VMEM((2 * NBUF, CHUNK), jnp.int32),
        pltpu.VMEM((2 * NBUF, CHUNK), jnp.int32),
    ]
    scratch += [pltpu.VMEM((CHUNK, D), jnp.float32) for _ in range(NBUF)]
    scratch += [pltpu.SemaphoreType.DMA for _ in range(2 * NBUF)]
    scratch.append(pltpu.VMEM_SHARED((NPAD, D), jnp.float32))
    return pl.kernel(
        functools.partial(_agg_body, with_cnt),
        out_type=tuple(out_type) if with_cnt else out_type[0],
        mesh=_MESH,
        scratch_types=scratch,
        name="sage_edge_agg" + ("_cnt" if with_cnt else ""),
    )


_ROWS_BLK = 1024


def _recip_body(cnt_ref, out_ref):
    cnt = cnt_ref[0, :, 0:1] + cnt_ref[1, :, 0:1]
    out_ref[...] = 1.0 / jnp.maximum(cnt, 1.0)


_recip_call = pl.pallas_call(
    _recip_body,
    grid=(pl.cdiv(N, _ROWS_BLK),),
    in_specs=[pl.BlockSpec((NC, _ROWS_BLK, D), lambda i: (0, i, 0))],
    out_specs=pl.BlockSpec((_ROWS_BLK, 1), lambda i: (i, 0)),
    out_shape=jax.ShapeDtypeStruct((N, 1), jnp.float32),
    name="sage_recip_cnt",
)


def _layer_body(relu_res, p_ref, recip_ref, h_ref, wl_ref, bl_ref, wr_ref,
                out_ref):
    mean = (p_ref[0] + p_ref[1]) * recip_ref[...]
    acc = jnp.dot(mean, wl_ref[...], preferred_element_type=jnp.float32)
    acc = acc + jnp.dot(h_ref[...], wr_ref[...], preferred_element_type=jnp.float32)
    acc = acc + bl_ref[...]
    if relu_res:
        acc = jnp.maximum(acc, 0.0) + h_ref[...]
    out_ref[...] = acc


def _make_layer(relu_res):
    return pl.pallas_call(
        functools.partial(_layer_body, relu_res),
        grid=(pl.cdiv(N, _ROWS_BLK),),
        in_specs=[
            pl.BlockSpec((NC, _ROWS_BLK, D), lambda i: (0, i, 0)),
            pl.BlockSpec((_ROWS_BLK, 1), lambda i: (i, 0)),
            pl.BlockSpec((_ROWS_BLK, D), lambda i: (i, 0)),
            pl.BlockSpec((D, D), lambda i: (0, 0)),
            pl.BlockSpec((1, D), lambda i: (0, 0)),
            pl.BlockSpec((D, D), lambda i: (0, 0)),
        ],
        out_specs=pl.BlockSpec((_ROWS_BLK, D), lambda i: (i, 0)),
        out_shape=jax.ShapeDtypeStruct((N, D), jnp.float32),
        name="sage_dense" + ("_relu_res" if relu_res else ""),
    )


def kernel(x, edge_index, W0l, b0l, W0r, W1l, b1l, W1r, W2l, b2l, W2r):
    e = edge_index.shape[1]
    pad = EPAD - e
    pad_src = jnp.zeros((pad,), jnp.int32)
    pad_dst = N + (jnp.arange(pad, dtype=jnp.int32) % (NPAD - N))
    src = jnp.concatenate([edge_index[0], pad_src])
    dst = jnp.concatenate([edge_index[1], pad_dst])
    s4 = src.reshape(NW, ROUNDS, NBUF, CHUNK)
    d4 = dst.reshape(NW, ROUNDS, NBUF, CHUNK)
    eidx = jnp.concatenate([s4, d4], axis=2).reshape(NW * ROUNDS * 2 * NBUF,
                                                     CHUNK)
    zrows = jnp.zeros((ROWS_PER_TILE, D), jnp.float32)
    ones_rows = jnp.ones((CHUNK, D), jnp.float32)

    agg_cnt = _make_agg(True)
    agg = _make_agg(False)
    layer_mid = _make_layer(True)
    layer_last = _make_layer(False)

    p0, cnt = agg_cnt(x, eidx, zrows, ones_rows)
    recip = _recip_call(cnt)
    h = layer_mid(p0, recip, x, W0l, b0l.reshape(1, D), W0r)
    p1 = agg(h, eidx, zrows)
    h = layer_mid(p1, recip, h, W1l, b1l.reshape(1, D), W1r)
    p2 = agg(h, eidx, zrows)
    h = layer_last(p2, recip, h, W2l, b2l.reshape(1, D), W2r)
    return h


# speedup vs baseline: 1.0302x; 1.0302x over previous
import functools

import jax
import jax.numpy as jnp
from jax import lax
from jax.experimental import pallas as pl
from jax.experimental.pallas import tpu as pltpu
from jax.experimental.pallas import tpu_sc as plsc

N = 10000
D = 128
NC = 2
NS = 16
NW = NC * NS
NPAD = 10112
ROWS_PER_TILE = NPAD // NS
CHUNK = 128
NBUF = 2
CPR = 8
ROUNDS = 10
EPW = ROUNDS * CPR * CHUNK
EPAD = NW * EPW

_MESH = plsc.VectorSubcoreMesh(core_axis_name="c", subcore_axis_name="s")


def _agg_body(with_cnt, *refs):
    if with_cnt:
        (h_hbm, eidx_hbm, zrows_hbm, ones_hbm, psum_hbm, cnt_hbm,
         idx0, idx1, *bufs) = refs
    else:
        (h_hbm, eidx_hbm, zrows_hbm, psum_hbm, idx0, idx1, *bufs) = refs
    rows = bufs[:NBUF]
    gsem = bufs[NBUF:2 * NBUF]
    acc_sh = bufs[2 * NBUF]
    c = lax.axis_index("c")
    s = lax.axis_index("s")
    wid = s * NC + c
    rs = s * ROWS_PER_TILE
    idxb = [idx0, idx1]
    ebase = (wid * ROUNDS) * (2 * CPR)

    pltpu.sync_copy(zrows_hbm, acc_sh.at[pl.ds(rs, ROWS_PER_TILE)])
    plsc.subcore_barrier()

    n_chunks = ROUNDS * CPR
    def idx_of(i):
        return idxb[(i // CPR) & 1], i % CPR

    pltpu.sync_copy(eidx_hbm.at[pl.ds(ebase, 2 * CPR)], idxb[0])
    iv0, k0 = idx_of(0)
    pend = pltpu.async_copy(h_hbm.at[iv0.at[k0]], rows[0], gsem[0])
    for i in range(n_chunks):
        b = i & 1
        iv, k = idx_of(i)
        nxt = i + 1
        if nxt < n_chunks:
            if nxt % CPR == 0:
                r = nxt // CPR
                pltpu.sync_copy(
                    eidx_hbm.at[pl.ds(ebase + r * 2 * CPR, 2 * CPR)],
                    idxb[r & 1])
            ivn, kn = idx_of(nxt)
            pend_next = pltpu.async_copy(h_hbm.at[ivn.at[kn]], rows[nxt & 1],
                                         gsem[nxt & 1])
        pend.wait()
        pltpu.sync_copy(rows[b], acc_sh.at[iv.at[CPR + k]], add=True)
        if nxt < n_chunks:
            pend = pend_next
    plsc.subcore_barrier()

    pltpu.sync_copy(acc_sh.at[pl.ds(rs, ROWS_PER_TILE)],
                    psum_hbm.at[c, pl.ds(rs, ROWS_PER_TILE)])

    if with_cnt:
        plsc.subcore_barrier()
        pltpu.sync_copy(zrows_hbm, acc_sh.at[pl.ds(rs, ROWS_PER_TILE)])
        pltpu.sync_copy(ones_hbm, rows[0])
        plsc.subcore_barrier()

        for r in range(ROUNDS):
            iv = idxb[r & 1]
            pltpu.sync_copy(
                eidx_hbm.at[pl.ds(ebase + r * 2 * CPR, 2 * CPR)], iv)
            for k in range(CPR):
                pltpu.sync_copy(rows[0], acc_sh.at[iv.at[CPR + k]], add=True)
        plsc.subcore_barrier()
        pltpu.sync_copy(acc_sh.at[pl.ds(rs, ROWS_PER_TILE)],
                        cnt_hbm.at[c, pl.ds(rs, ROWS_PER_TILE)])


def _make_agg(with_cnt):
    out_type = [jax.ShapeDtypeStruct((NC, NPAD, D), jnp.float32)]
    if with_cnt:
        out_type.append(jax.ShapeDtypeStruct((NC, NPAD, D), jnp.float32))
    scratch = [
        pltpu.VMEM((2 * CPR, CHUNK), jnp.int32),
        pltpu.VMEM((2 * CPR, CHUNK), jnp.int32),
    ]
    scratch += [pltpu.VMEM((CHUNK, D), jnp.float32) for _ in range(NBUF)]
    scratch += [pltpu.SemaphoreType.DMA for _ in range(NBUF)]
    scratch.append(pltpu.VMEM_SHARED((NPAD, D), jnp.float32))
    return pl.kernel(
        functools.partial(_agg_body, with_cnt),
        out_type=tuple(out_type) if with_cnt else out_type[0],
        mesh=_MESH,
        scratch_types=scratch,
        name="sage_edge_agg" + ("_cnt" if with_cnt else ""),
    )


_ROWS_BLK = 1024


def _recip_body(cnt_ref, out_ref):
    cnt = cnt_ref[0, :, 0:1] + cnt_ref[1, :, 0:1]
    out_ref[...] = 1.0 / jnp.maximum(cnt, 1.0)


_recip_call = pl.pallas_call(
    _recip_body,
    grid=(pl.cdiv(N, _ROWS_BLK),),
    in_specs=[pl.BlockSpec((NC, _ROWS_BLK, D), lambda i: (0, i, 0))],
    out_specs=pl.BlockSpec((_ROWS_BLK, 1), lambda i: (i, 0)),
    out_shape=jax.ShapeDtypeStruct((N, 1), jnp.float32),
    name="sage_recip_cnt",
)


def _layer_body(relu_res, p_ref, recip_ref, h_ref, wl_ref, bl_ref, wr_ref,
                out_ref):
    mean = (p_ref[0] + p_ref[1]) * recip_ref[...]
    acc = jnp.dot(mean, wl_ref[...], preferred_element_type=jnp.float32)
    acc = acc + jnp.dot(h_ref[...], wr_ref[...], preferred_element_type=jnp.float32)
    acc = acc + bl_ref[...]
    if relu_res:
        acc = jnp.maximum(acc, 0.0) + h_ref[...]
    out_ref[...] = acc


def _make_layer(relu_res):
    return pl.pallas_call(
        functools.partial(_layer_body, relu_res),
        grid=(pl.cdiv(N, _ROWS_BLK),),
        in_specs=[
            pl.BlockSpec((NC, _ROWS_BLK, D), lambda i: (0, i, 0)),
            pl.BlockSpec((_ROWS_BLK, 1), lambda i: (i, 0)),
            pl.BlockSpec((_ROWS_BLK, D), lambda i: (i, 0)),
            pl.BlockSpec((D, D), lambda i: (0, 0)),
            pl.BlockSpec((1, D), lambda i: (0, 0)),
            pl.BlockSpec((D, D), lambda i: (0, 0)),
        ],
        out_specs=pl.BlockSpec((_ROWS_BLK, D), lambda i: (i, 0)),
        out_shape=jax.ShapeDtypeStruct((N, D), jnp.float32),
        name="sage_dense" + ("_relu_res" if relu_res else ""),
    )


def kernel(x, edge_index, W0l, b0l, W0r, W1l, b1l, W1r, W2l, b2l, W2r):
    e = edge_index.shape[1]
    pad = EPAD - e
    pad_src = jnp.zeros((pad,), jnp.int32)
    pad_dst = N + (jnp.arange(pad, dtype=jnp.int32) % (NPAD - N))
    src = jnp.concatenate([edge_index[0], pad_src])
    dst = jnp.concatenate([edge_index[1], pad_dst])
    s4 = src.reshape(NW, ROUNDS, CPR, CHUNK)
    d4 = dst.reshape(NW, ROUNDS, CPR, CHUNK)
    eidx = jnp.concatenate([s4, d4], axis=2).reshape(NW * ROUNDS * 2 * CPR,
                                                     CHUNK)
    zrows = jnp.zeros((ROWS_PER_TILE, D), jnp.float32)
    ones_rows = jnp.ones((CHUNK, D), jnp.float32)

    agg_cnt = _make_agg(True)
    agg = _make_agg(False)
    layer_mid = _make_layer(True)
    layer_last = _make_layer(False)

    p0, cnt = agg_cnt(x, eidx, zrows, ones_rows)
    recip = _recip_call(cnt)
    h = layer_mid(p0, recip, x, W0l, b0l.reshape(1, D), W0r)
    p1 = agg(h, eidx, zrows)
    h = layer_mid(p1, recip, h, W1l, b1l.reshape(1, D), W1r)
    p2 = agg(h, eidx, zrows)
    h = layer_last(p2, recip, h, W2l, b2l.reshape(1, D), W2r)
    return h
